# Initial kernel scaffold; baseline (speedup 1.0000x reference)
import jax
import jax.numpy as jnp
from jax.experimental import pallas as pl


def kernel(input_, weight):
    raise NotImplementedError("write your pallas kernel here")


# speedup vs baseline: 1.0939x; 1.0939x over previous
import functools

import jax
import jax.numpy as jnp
from jax import lax
from jax.experimental import pallas as pl
from jax.experimental.pallas import tpu as pltpu
from jax.experimental.pallas import tpu_sc as plsc

_NUM_CORES = 2
_NUM_SUBCORES = 16
_NUM_WORKERS = _NUM_CORES * _NUM_SUBCORES
_WINDOW = 1024


def _gather_rows(weight, idx_flat, num_idx, dim):
    b_per_w = num_idx // _NUM_WORKERS
    mesh = plsc.VectorSubcoreMesh(core_axis_name="c", subcore_axis_name="s")

    @functools.partial(
        pl.kernel,
        mesh=mesh,
        out_type=jax.ShapeDtypeStruct((num_idx, dim), weight.dtype),
        scratch_types=[
            pltpu.VMEM((_WINDOW,), jnp.int32),
            pltpu.VMEM((_WINDOW, dim), weight.dtype),
            pltpu.SemaphoreType.DMA,
        ],
        compiler_params=pltpu.CompilerParams(use_tc_tiling_on_sc=False),
    )
    def k(table_hbm, idx_hbm, out_hbm, idx_v, rows_v, sem):
        wid = lax.axis_index("s") * _NUM_CORES + lax.axis_index("c")
        base = wid * b_per_w

        @pl.loop(0, b_per_w, step=_WINDOW)
        def _(off):
            pltpu.sync_copy(idx_hbm.at[pl.ds(base + off, _WINDOW)], idx_v)
            pltpu.async_copy(table_hbm.at[idx_v], rows_v, sem).wait()
            pltpu.sync_copy(rows_v, out_hbm.at[pl.ds(base + off, _WINDOW)])

    return k(weight, idx_flat)


def kernel(input_, weight):
    batch, hist = input_.shape
    num_idx = batch * hist
    dim = weight.shape[1]
    idx_flat = input_.astype(jnp.int32).reshape(num_idx)
    out = _gather_rows(weight, idx_flat, num_idx, dim)
    return out.reshape(batch, hist, dim)


# speedup vs baseline: 1.1089x; 1.0138x over previous
import functools

import jax
import jax.numpy as jnp
from jax import lax
from jax.experimental import pallas as pl
from jax.experimental.pallas import tpu as pltpu
from jax.experimental.pallas import tpu_sc as plsc

_NUM_CORES = 2
_NUM_SUBCORES = 16
_NUM_WORKERS = _NUM_CORES * _NUM_SUBCORES
_WINDOW = 1280


def _gather_rows(weight, idx_flat, num_idx, dim):
    b_per_w = num_idx // _NUM_WORKERS
    steps = b_per_w // _WINDOW
    assert steps % 2 == 0 and steps * _WINDOW == b_per_w
    mesh = plsc.VectorSubcoreMesh(core_axis_name="c", subcore_axis_name="s")
    W = _WINDOW

    @functools.partial(
        pl.kernel,
        mesh=mesh,
        out_type=jax.ShapeDtypeStruct((num_idx, dim), weight.dtype),
        scratch_types=[
            pltpu.VMEM((W,), jnp.int32),
            pltpu.VMEM((W,), jnp.int32),
            pltpu.VMEM((W, dim), weight.dtype),
            pltpu.VMEM((W, dim), weight.dtype),
            pltpu.SemaphoreType.DMA,
            pltpu.SemaphoreType.DMA,
            pltpu.SemaphoreType.DMA,
            pltpu.SemaphoreType.DMA,
            pltpu.SemaphoreType.DMA,
            pltpu.SemaphoreType.DMA,
        ],
        compiler_params=pltpu.CompilerParams(use_tc_tiling_on_sc=False),
    )
    def k(table_hbm, idx_hbm, out_hbm,
          i0, i1, r0, r1, si0, si1, sg0, sg1, sw0, sw1):
        wid = lax.axis_index("s") * _NUM_CORES + lax.axis_index("c")
        base = wid * b_per_w
        bufs = ((i0, si0, r0, sg0, sw0), (i1, si1, r1, sg1, sw1))

        pltpu.async_copy(idx_hbm.at[pl.ds(base, W)], i0, si0)
        pltpu.async_copy(idx_hbm.at[pl.ds(base + W, W)], i1, si1)

        @pl.loop(0, steps, step=2)
        def _(t):
            for b, (iv, si, rv, sg, sw) in enumerate(bufs):
                off = base + (t + b) * W
                pltpu.make_async_copy(idx_hbm.at[pl.ds(off, W)], iv, si).wait()

                @pl.when(t + b >= 2)
                def _():
                    pltpu.make_async_copy(
                        rv, out_hbm.at[pl.ds(off - 2 * W, W)], sw).wait()

                pltpu.async_copy(table_hbm.at[iv], rv, sg)

            for b, (iv, si, rv, sg, sw) in enumerate(bufs):
                off = base + (t + b) * W
                pltpu.make_async_copy(table_hbm.at[iv], rv, sg).wait()

                @pl.when(t + b + 2 < steps)
                def _():
                    pltpu.async_copy(
                        idx_hbm.at[pl.ds(off + 2 * W, W)], iv, si)

                pltpu.async_copy(rv, out_hbm.at[pl.ds(off, W)], sw)

        pltpu.make_async_copy(
            r0, out_hbm.at[pl.ds(base + (steps - 2) * W, W)], sw0).wait()
        pltpu.make_async_copy(
            r1, out_hbm.at[pl.ds(base + (steps - 1) * W, W)], sw1).wait()

    return k(weight, idx_flat)


def kernel(input_, weight):
    batch, hist = input_.shape
    num_idx = batch * hist
    dim = weight.shape[1]
    idx_flat = input_.astype(jnp.int32).reshape(num_idx)
    out = _gather_rows(weight, idx_flat, num_idx, dim)
    return out.reshape(batch, hist, dim)


# speedup vs baseline: 1.7952x; 1.6189x over previous
import functools

import jax
import jax.numpy as jnp
from jax import lax
from jax.experimental import pallas as pl
from jax.experimental.pallas import tpu as pltpu
from jax.experimental.pallas import tpu_sc as plsc

_NUM_CORES = 2
_NUM_SUBCORES = 16
_NUM_WORKERS = _NUM_CORES * _NUM_SUBCORES
_WB = 16


def _gather(weight, idx, batch, hist, dim):
    rows_per_w = batch // _NUM_WORKERS
    steps = rows_per_w // _WB
    assert steps % 2 == 0 and steps * _WB == rows_per_w
    mesh = plsc.VectorSubcoreMesh(core_axis_name="c", subcore_axis_name="s")

    @functools.partial(
        pl.kernel,
        mesh=mesh,
        out_type=jax.ShapeDtypeStruct((batch, hist, dim), weight.dtype),
        scratch_types=[
            pltpu.VMEM((_WB, hist), jnp.int32),
            pltpu.VMEM((_WB, hist), jnp.int32),
            pltpu.VMEM((_WB, hist, dim), weight.dtype),
            pltpu.VMEM((_WB, hist, dim), weight.dtype),
            pltpu.SemaphoreType.DMA,
            pltpu.SemaphoreType.DMA,
            pltpu.SemaphoreType.DMA,
            pltpu.SemaphoreType.DMA,
            pltpu.SemaphoreType.DMA,
            pltpu.SemaphoreType.DMA,
        ],
        compiler_params=pltpu.CompilerParams(use_tc_tiling_on_sc=False),
    )
    def k(table_hbm, idx_hbm, out_hbm,
          i0, i1, r0, r1, si0, si1, sg0, sg1, sw0, sw1):
        wid = lax.axis_index("s") * _NUM_CORES + lax.axis_index("c")
        base = wid * rows_per_w
        bufs = ((i0, si0, r0, sg0, sw0), (i1, si1, r1, sg1, sw1))

        pltpu.async_copy(idx_hbm.at[pl.ds(base, _WB)], i0, si0)
        pltpu.async_copy(idx_hbm.at[pl.ds(base + _WB, _WB)], i1, si1)

        @pl.loop(0, steps, step=2)
        def _(t):
            for b, (iv, si, rv, sg, sw) in enumerate(bufs):
                row = base + (t + b) * _WB
                pltpu.make_async_copy(
                    idx_hbm.at[pl.ds(row, _WB)], iv, si).wait()

                @pl.when(t + b >= 2)
                def _():
                    pltpu.make_async_copy(
                        rv, out_hbm.at[pl.ds(row - 2 * _WB, _WB)], sw).wait()

                for r in range(_WB):
                    pltpu.async_copy(table_hbm.at[iv.at[r]], rv.at[r], sg)

            for b, (iv, si, rv, sg, sw) in enumerate(bufs):
                row = base + (t + b) * _WB
                for r in range(_WB):
                    pltpu.make_async_copy(
                        table_hbm.at[iv.at[r]], rv.at[r], sg).wait()

                @pl.when(t + b + 2 < steps)
                def _():
                    pltpu.async_copy(
                        idx_hbm.at[pl.ds(row + 2 * _WB, _WB)], iv, si)

                pltpu.async_copy(rv, out_hbm.at[pl.ds(row, _WB)], sw)

        pltpu.make_async_copy(
            r0, out_hbm.at[pl.ds(base + (steps - 2) * _WB, _WB)], sw0).wait()
        pltpu.make_async_copy(
            r1, out_hbm.at[pl.ds(base + (steps - 1) * _WB, _WB)], sw1).wait()

    return k(weight, idx)


def kernel(input_, weight):
    batch, hist = input_.shape
    dim = weight.shape[1]
    return _gather(weight, input_.astype(jnp.int32), batch, hist, dim)
